# baseline (speedup 1.0000x reference)
import jax
import jax.numpy as jnp
from jax import lax
from jax.experimental import pallas as pl
from jax.experimental.pallas import tpu as pltpu
from jax.experimental.pallas import tpu_sc as plsc

B = 16384
L = 200
D = 64
N = 1_000_000
LP = 208
NEG = -1e9

_BK = 8192


def _score_table_body(ntt_ref, w1_ref, b1_ref, w2_ref, b2_ref, out_ref):
    i = pl.program_id(0)
    x = ntt_ref[...]
    a = jnp.tanh(
        lax.dot_general(w1_ref[...], x, (((1,), (0,)), ((), ())),
                        preferred_element_type=jnp.float32)
        + b1_ref[...][:, None])
    g = lax.dot_general(w2_ref[...], a, (((1,), (0,)), ((), ())),
                        preferred_element_type=jnp.float32)
    g = (g + b2_ref[...][None, :]).reshape(_BK)
    nidx = lax.broadcasted_iota(jnp.int32, (_BK,), 0) + i * _BK
    out_ref[...] = jnp.where(nidx == 0, jnp.float32(NEG), g)


def _score_table(news_t, W_a1, b_a1, W_a2, b_a2):
    grid = pl.cdiv(N, _BK)
    return pl.pallas_call(
        _score_table_body,
        grid=(grid,),
        in_specs=[
            pl.BlockSpec((D, _BK), lambda i: (0, i)),
            pl.BlockSpec((D // 2, D), lambda i: (0, 0)),
            pl.BlockSpec((D // 2,), lambda i: (0,)),
            pl.BlockSpec((1, D // 2), lambda i: (0, 0)),
            pl.BlockSpec((1,), lambda i: (0,)),
        ],
        out_specs=pl.BlockSpec((_BK,), lambda i: (i,)),
        out_shape=jax.ShapeDtypeStruct((N,), jnp.float32),
    )(news_t, W_a1, b_a1, W_a2, b_a2)


def _sc_body(hist_hbm, g_hbm, news_hbm, uidx_hbm, utf_hbm, nidx_hbm,
             hnum_hbm, den_hbm, uet_hbm, iemb_hbm,
             hist_v, s_v, w_v, rows_v, out_v, den_v, uet_v, idx_v, idx2_v,
             sem):
    nc = 2
    wid = lax.axis_index("s") * nc + lax.axis_index("c")
    bpw = B // 32
    b0 = wid * bpw
    cc = 16

    zf = jnp.zeros((16,), jnp.float32)
    s_v[pl.ds(192, 16)] = jnp.full((16,), NEG, jnp.float32)
    for r in range(L, LP):
        for q in range(D // 16):
            rows_v[r, pl.ds(16 * q, 16)] = zf

    @pl.loop(0, bpw // cc)
    def _chunk(ci):
        pltpu.sync_copy(hist_hbm.at[pl.ds(b0 + ci * cc, cc), :], hist_v)

        @pl.loop(0, cc)
        def _per_b(r):
            bl = ci * cc + r
            c1 = pltpu.async_copy(g_hbm.at[hist_v.at[r, pl.ds(0, 104)]],
                                  s_v.at[pl.ds(0, 104)], sem)
            c2 = pltpu.async_copy(g_hbm.at[hist_v.at[r, pl.ds(104, 96)]],
                                  s_v.at[pl.ds(104, 96)], sem)
            c3 = pltpu.async_copy(news_hbm.at[hist_v.at[r, pl.ds(0, 104)]],
                                  rows_v.at[pl.ds(0, 104), :], sem)
            c4 = pltpu.async_copy(news_hbm.at[hist_v.at[r, pl.ds(104, 96)]],
                                  rows_v.at[pl.ds(104, 96), :], sem)
            c1.wait()
            c2.wait()
            den = jnp.zeros((16,), jnp.float32)
            for k in range(LP // 16):
                e = jnp.exp(s_v[pl.ds(16 * k, 16)])
                w_v[pl.ds(16 * k, 16)] = e
                den = den + e
            den_v_row = den
            c3.wait()
            c4.wait()

            z = jnp.zeros((16,), jnp.float32)

            @pl.loop(0, LP // 16, init_carry=(z, z, z, z))
            def _acc(k, carry):
                wv = w_v[pl.ds(16 * k, 16)]
                for j in range(16):
                    wl = wv[j]
                    l = 16 * k + j
                    carry = tuple(c + rows_v[l, pl.ds(16 * q, 16)] * wl
                                  for q, c in enumerate(carry))
                return carry

            a0, a1, a2, a3 = _acc
            out_v[bl, pl.ds(0, 16)] = a0
            out_v[bl, pl.ds(16, 16)] = a1
            out_v[bl, pl.ds(32, 16)] = a2
            out_v[bl, pl.ds(48, 16)] = a3
            den_v[bl, :] = den_v_row

    pltpu.sync_copy(out_v, hnum_hbm.at[pl.ds(b0, bpw)])
    pltpu.sync_copy(den_v, den_hbm.at[pl.ds(b0, bpw)])

    pltpu.sync_copy(nidx_hbm.at[pl.ds(b0, bpw)], idx_v)
    for j in range(bpw // 128):
        pltpu.async_copy(news_hbm.at[idx_v.at[pl.ds(j * 128, 128)]],
                         out_v.at[pl.ds(j * 128, 128), :], sem).wait()
    pltpu.sync_copy(out_v, iemb_hbm.at[pl.ds(b0, bpw)])

    pltpu.sync_copy(uidx_hbm.at[pl.ds(b0, bpw)], idx_v)

    @pl.loop(0, D)
    def _plane(d):
        off = d * N
        for j in range(bpw // 16):
            idx2_v[pl.ds(16 * j, 16)] = idx_v[pl.ds(16 * j, 16)] + off
        for j in range(bpw // 128):
            pltpu.async_copy(utf_hbm.at[idx2_v.at[pl.ds(j * 128, 128)]],
                             uet_v.at[d, pl.ds(j * 128, 128)], sem).wait()

    pltpu.sync_copy(uet_v, uet_hbm.at[:, pl.ds(b0, bpw)])


def _sc_gather(history, g, news_table, user_idx, user_flat, news_idx):
    bpw = B // 32
    mesh = plsc.VectorSubcoreMesh(core_axis_name="c", subcore_axis_name="s")
    f = pl.kernel(
        _sc_body,
        out_type=(
            jax.ShapeDtypeStruct((B, D), jnp.float32),
            jax.ShapeDtypeStruct((B, 16), jnp.float32),
            jax.ShapeDtypeStruct((D, B), jnp.float32),
            jax.ShapeDtypeStruct((B, D), jnp.float32),
        ),
        mesh=mesh,
        scratch_types=[
            pltpu.VMEM((16, L), jnp.int32),
            pltpu.VMEM((LP,), jnp.float32),
            pltpu.VMEM((LP,), jnp.float32),
            pltpu.VMEM((LP, D), jnp.float32),
            pltpu.VMEM((bpw, D), jnp.float32),
            pltpu.VMEM((bpw, 16), jnp.float32),
            pltpu.VMEM((D, bpw), jnp.float32),
            pltpu.VMEM((bpw,), jnp.int32),
            pltpu.VMEM((bpw,), jnp.int32),
            pltpu.SemaphoreType.DMA,
        ],
        compiler_params=pltpu.CompilerParams(use_tc_tiling_on_sc=False),
    )
    return f(history, g, news_table, user_idx, user_flat, news_idx)


_RB = 2048


def _final_body(uet_ref, hn_ref, den_ref, ie_ref, wut_ref, but_ref, wnt_ref,
                bnt_ref, out_ref):
    den = jnp.sum(den_ref[...], axis=1, keepdims=True)
    hr = hn_ref[...] * jnp.where(den > 0, 1.0 / den, 0.0)
    uW = lax.dot_general(uet_ref[...], wut_ref[...], (((0,), (1,)), ((), ())),
                         preferred_element_type=jnp.float32)
    hW = lax.dot_general(hr, wut_ref[...], (((1,), (1,)), ((), ())),
                         preferred_element_type=jnp.float32)
    ur = jax.nn.relu(uW + hW + but_ref[...][None, :])
    nr = jax.nn.relu(
        lax.dot_general(ie_ref[...], wnt_ref[...], (((1,), (1,)), ((), ())),
                        preferred_element_type=jnp.float32)
        + bnt_ref[...][None, :])
    out_ref[...] = jax.nn.sigmoid(jnp.sum(ur * nr, axis=1))


def _final(uet, hist_num, den, id_emb, W_ut, b_ut, W_nt, b_nt):
    grid = B // _RB
    return pl.pallas_call(
        _final_body,
        grid=(grid,),
        in_specs=[
            pl.BlockSpec((D, _RB), lambda i: (0, i)),
            pl.BlockSpec((_RB, D), lambda i: (i, 0)),
            pl.BlockSpec((_RB, 16), lambda i: (i, 0)),
            pl.BlockSpec((_RB, D), lambda i: (i, 0)),
            pl.BlockSpec((D, D), lambda i: (0, 0)),
            pl.BlockSpec((D,), lambda i: (0,)),
            pl.BlockSpec((D, D), lambda i: (0, 0)),
            pl.BlockSpec((D,), lambda i: (0,)),
        ],
        out_specs=pl.BlockSpec((_RB,), lambda i: (i,)),
        out_shape=jax.ShapeDtypeStruct((B,), jnp.float32),
    )(uet, hist_num, den, id_emb, W_ut, b_ut, W_nt, b_nt)


def kernel(user_idx, news_idx, history, user_table, news_table,
           W_ut, b_ut, W_nt, b_nt, W_a1, b_a1, W_a2, b_a2):
    news_t = news_table.T
    user_flat = user_table.T.reshape(D * N)
    g = _score_table(news_t, W_a1, b_a1, W_a2, b_a2)
    hist_num, den, uet, id_emb = _sc_gather(
        history, g, news_table, user_idx, user_flat, news_idx)
    return _final(uet, hist_num, den, id_emb, W_ut, b_ut, W_nt, b_nt)


# speedup vs baseline: 3.1100x; 3.1100x over previous
import jax
import jax.numpy as jnp
from jax import lax
from jax.experimental import pallas as pl
from jax.experimental.pallas import tpu as pltpu
from jax.experimental.pallas import tpu_sc as plsc

B = 16384
L = 200
D = 64
N = 1_000_000
LP = 208
NEG = -1e9

_BK = 8192


def _score_table_body(ntt_ref, w1_ref, b1_ref, w2_ref, b2_ref, out_ref):
    i = pl.program_id(0)
    x = ntt_ref[...]
    a = jnp.tanh(
        lax.dot_general(w1_ref[...], x, (((1,), (0,)), ((), ())),
                        preferred_element_type=jnp.float32)
        + b1_ref[...][:, None])
    g = lax.dot_general(w2_ref[...], a, (((1,), (0,)), ((), ())),
                        preferred_element_type=jnp.float32)
    g = (g + b2_ref[...][None, :]).reshape(_BK)
    nidx = lax.broadcasted_iota(jnp.int32, (_BK,), 0) + i * _BK
    out_ref[...] = jnp.where(nidx == 0, jnp.float32(NEG), g)


def _score_table(news_t, W_a1, b_a1, W_a2, b_a2):
    grid = pl.cdiv(N, _BK)
    return pl.pallas_call(
        _score_table_body,
        grid=(grid,),
        in_specs=[
            pl.BlockSpec((D, _BK), lambda i: (0, i)),
            pl.BlockSpec((D // 2, D), lambda i: (0, 0)),
            pl.BlockSpec((D // 2,), lambda i: (0,)),
            pl.BlockSpec((1, D // 2), lambda i: (0, 0)),
            pl.BlockSpec((1,), lambda i: (0,)),
        ],
        out_specs=pl.BlockSpec((_BK,), lambda i: (i,)),
        out_shape=jax.ShapeDtypeStruct((N,), jnp.float32),
    )(news_t, W_a1, b_a1, W_a2, b_a2)


def _sc_body(hist_hbm, g_hbm, news_hbm, uidx_hbm, utab_hbm, nidx_hbm,
             hnum_hbm, den_hbm, uemb_hbm, iemb_hbm,
             hist_v, s_v, w_v, rows_v, out_v, den_v, idx_v, sem):
    nc = 2
    wid = lax.axis_index("s") * nc + lax.axis_index("c")
    bpw = B // 32
    b0 = wid * bpw
    cc = 16

    zf = jnp.zeros((16,), jnp.float32)
    s_v[pl.ds(192, 16)] = jnp.full((16,), NEG, jnp.float32)
    for r in range(L, LP):
        for q in range(D // 16):
            rows_v[r, pl.ds(16 * q, 16)] = zf

    @pl.loop(0, bpw // cc)
    def _chunk(ci):
        pltpu.sync_copy(hist_hbm.at[pl.ds(b0 + ci * cc, cc), :], hist_v)

        @pl.loop(0, cc)
        def _per_b(r):
            bl = ci * cc + r
            c1 = pltpu.async_copy(g_hbm.at[hist_v.at[r, pl.ds(0, 104)]],
                                  s_v.at[pl.ds(0, 104)], sem)
            c2 = pltpu.async_copy(g_hbm.at[hist_v.at[r, pl.ds(104, 96)]],
                                  s_v.at[pl.ds(104, 96)], sem)
            c3 = pltpu.async_copy(news_hbm.at[hist_v.at[r, pl.ds(0, 104)]],
                                  rows_v.at[pl.ds(0, 104), :], sem)
            c4 = pltpu.async_copy(news_hbm.at[hist_v.at[r, pl.ds(104, 96)]],
                                  rows_v.at[pl.ds(104, 96), :], sem)
            c1.wait()
            c2.wait()
            den = jnp.zeros((16,), jnp.float32)
            for k in range(LP // 16):
                e = jnp.exp(s_v[pl.ds(16 * k, 16)])
                w_v[pl.ds(16 * k, 16)] = e
                den = den + e
            den_v_row = den
            c3.wait()
            c4.wait()

            z = jnp.zeros((16,), jnp.float32)

            @pl.loop(0, LP // 16, init_carry=(z, z, z, z))
            def _acc(k, carry):
                wv = w_v[pl.ds(16 * k, 16)]
                for j in range(16):
                    wl = wv[j]
                    l = 16 * k + j
                    carry = tuple(c + rows_v[l, pl.ds(16 * q, 16)] * wl
                                  for q, c in enumerate(carry))
                return carry

            a0, a1, a2, a3 = _acc
            out_v[bl, pl.ds(0, 16)] = a0
            out_v[bl, pl.ds(16, 16)] = a1
            out_v[bl, pl.ds(32, 16)] = a2
            out_v[bl, pl.ds(48, 16)] = a3
            den_v[bl, :] = den_v_row

    pltpu.sync_copy(out_v, hnum_hbm.at[pl.ds(b0, bpw)])
    pltpu.sync_copy(den_v, den_hbm.at[pl.ds(b0, bpw)])

    pltpu.sync_copy(nidx_hbm.at[pl.ds(b0, bpw)], idx_v)
    for j in range(bpw // 128):
        pltpu.async_copy(news_hbm.at[idx_v.at[pl.ds(j * 128, 128)]],
                         out_v.at[pl.ds(j * 128, 128), :], sem).wait()
    pltpu.sync_copy(out_v, iemb_hbm.at[pl.ds(b0, bpw)])

    pltpu.sync_copy(uidx_hbm.at[pl.ds(b0, bpw)], idx_v)
    for j in range(bpw // 128):
        pltpu.async_copy(utab_hbm.at[idx_v.at[pl.ds(j * 128, 128)]],
                         out_v.at[pl.ds(j * 128, 128), :], sem).wait()
    pltpu.sync_copy(out_v, uemb_hbm.at[pl.ds(b0, bpw)])


def _sc_gather(history, g, news_table, user_idx, user_table, news_idx):
    bpw = B // 32
    mesh = plsc.VectorSubcoreMesh(core_axis_name="c", subcore_axis_name="s")
    f = pl.kernel(
        _sc_body,
        out_type=(
            jax.ShapeDtypeStruct((B, D), jnp.float32),
            jax.ShapeDtypeStruct((B, 16), jnp.float32),
            jax.ShapeDtypeStruct((B, D), jnp.float32),
            jax.ShapeDtypeStruct((B, D), jnp.float32),
        ),
        mesh=mesh,
        scratch_types=[
            pltpu.VMEM((16, L), jnp.int32),
            pltpu.VMEM((LP,), jnp.float32),
            pltpu.VMEM((LP,), jnp.float32),
            pltpu.VMEM((LP, D), jnp.float32),
            pltpu.VMEM((bpw, D), jnp.float32),
            pltpu.VMEM((bpw, 16), jnp.float32),
            pltpu.VMEM((bpw,), jnp.int32),
            pltpu.SemaphoreType.DMA,
        ],
        compiler_params=pltpu.CompilerParams(use_tc_tiling_on_sc=False),
    )
    return f(history, g, news_table, user_idx, user_table, news_idx)


_RB = 2048


def _final_body(ue_ref, hn_ref, den_ref, ie_ref, wut_ref, but_ref, wnt_ref,
                bnt_ref, out_ref):
    den = jnp.sum(den_ref[...], axis=1, keepdims=True)
    hr = hn_ref[...] * jnp.where(den > 0, 1.0 / den, 0.0)
    u = ue_ref[...] + hr
    ur = jax.nn.relu(
        lax.dot_general(u, wut_ref[...], (((1,), (1,)), ((), ())),
                        preferred_element_type=jnp.float32)
        + but_ref[...][None, :])
    nr = jax.nn.relu(
        lax.dot_general(ie_ref[...], wnt_ref[...], (((1,), (1,)), ((), ())),
                        preferred_element_type=jnp.float32)
        + bnt_ref[...][None, :])
    out_ref[...] = jax.nn.sigmoid(jnp.sum(ur * nr, axis=1))


def _final(user_emb, hist_num, den, id_emb, W_ut, b_ut, W_nt, b_nt):
    grid = B // _RB
    return pl.pallas_call(
        _final_body,
        grid=(grid,),
        in_specs=[
            pl.BlockSpec((_RB, D), lambda i: (i, 0)),
            pl.BlockSpec((_RB, D), lambda i: (i, 0)),
            pl.BlockSpec((_RB, 16), lambda i: (i, 0)),
            pl.BlockSpec((_RB, D), lambda i: (i, 0)),
            pl.BlockSpec((D, D), lambda i: (0, 0)),
            pl.BlockSpec((D,), lambda i: (0,)),
            pl.BlockSpec((D, D), lambda i: (0, 0)),
            pl.BlockSpec((D,), lambda i: (0,)),
        ],
        out_specs=pl.BlockSpec((_RB,), lambda i: (i,)),
        out_shape=jax.ShapeDtypeStruct((B,), jnp.float32),
    )(user_emb, hist_num, den, id_emb, W_ut, b_ut, W_nt, b_nt)


def kernel(user_idx, news_idx, history, user_table, news_table,
           W_ut, b_ut, W_nt, b_nt, W_a1, b_a1, W_a2, b_a2):
    news_t = news_table.T
    g = _score_table(news_t, W_a1, b_a1, W_a2, b_a2)
    hist_num, den, user_emb, id_emb = _sc_gather(
        history, g, news_table, user_idx, user_table, news_idx)
    return _final(user_emb, hist_num, den, id_emb, W_ut, b_ut, W_nt, b_nt)


# speedup vs baseline: 4.5021x; 1.4476x over previous
import jax
import jax.numpy as jnp
from jax import lax
from jax.experimental import pallas as pl
from jax.experimental.pallas import tpu as pltpu
from jax.experimental.pallas import tpu_sc as plsc

B = 16384
L = 200
D = 64
N = 1_000_000
LP = 208
NEG = -1e9

_BK = 8192


def _score_table_body(ntt_ref, w1_ref, b1_ref, w2_ref, b2_ref, out_ref):
    i = pl.program_id(0)
    x = ntt_ref[...]
    a = jnp.tanh(
        lax.dot_general(w1_ref[...], x, (((1,), (0,)), ((), ())),
                        preferred_element_type=jnp.float32)
        + b1_ref[...][:, None])
    g = lax.dot_general(w2_ref[...], a, (((1,), (0,)), ((), ())),
                        preferred_element_type=jnp.float32)
    g = (g + b2_ref[...][None, :]).reshape(_BK)
    nidx = lax.broadcasted_iota(jnp.int32, (_BK,), 0) + i * _BK
    out_ref[...] = jnp.where(nidx == 0, jnp.float32(NEG), g)


def _score_table(news_t, W_a1, b_a1, W_a2, b_a2):
    grid = pl.cdiv(N, _BK)
    return pl.pallas_call(
        _score_table_body,
        grid=(grid,),
        in_specs=[
            pl.BlockSpec((D, _BK), lambda i: (0, i)),
            pl.BlockSpec((D // 2, D), lambda i: (0, 0)),
            pl.BlockSpec((D // 2,), lambda i: (0,)),
            pl.BlockSpec((1, D // 2), lambda i: (0, 0)),
            pl.BlockSpec((1,), lambda i: (0,)),
        ],
        out_specs=pl.BlockSpec((_BK,), lambda i: (i,)),
        out_shape=jax.ShapeDtypeStruct((N,), jnp.float32),
    )(news_t, W_a1, b_a1, W_a2, b_a2)


def _sc_body(hist_hbm, g_hbm, news_hbm, nidx_hbm,
             hnum_hbm, den_hbm, iemb_hbm,
             hist_v, s2_v, w_v, rows_v, out_v, den_v, idx_v,
             sem_s, sem_r, sem_h):
    nc = 2
    wid = lax.axis_index("s") * nc + lax.axis_index("c")
    bpw = B // 32
    b0 = wid * bpw
    cc = 16

    zf = jnp.zeros((16,), jnp.float32)
    for r in range(cc):
        s2_v[r, pl.ds(192, 16)] = jnp.full((16,), NEG, jnp.float32)
    for nb in range(2):
        for r in range(L, LP):
            for q in range(D // 16):
                rows_v[nb, r, pl.ds(16 * q, 16)] = zf

    def _issue_rows(hrow, nb):
        c1 = pltpu.async_copy(news_hbm.at[hist_v.at[hrow, pl.ds(0, 104)]],
                              rows_v.at[nb, pl.ds(0, 104), :], sem_r)
        c2 = pltpu.async_copy(news_hbm.at[hist_v.at[hrow, pl.ds(104, 96)]],
                              rows_v.at[nb, pl.ds(104, 96), :], sem_r)
        return c1, c2

    @pl.loop(0, bpw // cc)
    def _chunk(ci):
        pltpu.async_copy(hist_hbm.at[pl.ds(b0 + ci * cc, cc), :], hist_v,
                         sem_h).wait()
        sdescs = []
        for r in range(cc):
            sdescs.append(
                pltpu.async_copy(g_hbm.at[hist_v.at[r, pl.ds(0, 104)]],
                                 s2_v.at[r, pl.ds(0, 104)], sem_s))
            sdescs.append(
                pltpu.async_copy(g_hbm.at[hist_v.at[r, pl.ds(104, 96)]],
                                 s2_v.at[r, pl.ds(104, 96)], sem_s))
        prev = _issue_rows(0, 0)
        for r in range(cc):
            bl = ci * cc + r
            nxt = _issue_rows(r + 1, (r + 1) % 2) if r + 1 < cc else None
            sdescs[2 * r].wait()
            sdescs[2 * r + 1].wait()
            den = jnp.zeros((16,), jnp.float32)
            for k in range(LP // 16):
                e = jnp.exp(s2_v[r, pl.ds(16 * k, 16)])
                w_v[pl.ds(16 * k, 16)] = e
                den = den + e
            prev[0].wait()
            prev[1].wait()
            nb = r % 2

            z = jnp.zeros((16,), jnp.float32)

            @pl.loop(0, LP // 16, init_carry=(z, z, z, z))
            def _acc(k, carry):
                wv = w_v[pl.ds(16 * k, 16)]
                for j in range(16):
                    wl = wv[j]
                    l = 16 * k + j
                    carry = tuple(c + rows_v[nb, l, pl.ds(16 * q, 16)] * wl
                                  for q, c in enumerate(carry))
                return carry

            a0, a1, a2, a3 = _acc
            out_v[bl, pl.ds(0, 16)] = a0
            out_v[bl, pl.ds(16, 16)] = a1
            out_v[bl, pl.ds(32, 16)] = a2
            out_v[bl, pl.ds(48, 16)] = a3
            den_v[bl, :] = den
            prev = nxt

    pltpu.sync_copy(out_v, hnum_hbm.at[pl.ds(b0, bpw)])
    pltpu.sync_copy(den_v, den_hbm.at[pl.ds(b0, bpw)])

    pltpu.sync_copy(nidx_hbm.at[pl.ds(b0, bpw)], idx_v)
    for j in range(bpw // 128):
        pltpu.async_copy(news_hbm.at[idx_v.at[pl.ds(j * 128, 128)]],
                         out_v.at[pl.ds(j * 128, 128), :], sem_r).wait()
    pltpu.sync_copy(out_v, iemb_hbm.at[pl.ds(b0, bpw)])


def _user_body(uidx_hbm, utab_hbm, uemb_hbm, out_v, idx_v, sem):
    nc = 2
    wid = lax.axis_index("s") * nc + lax.axis_index("c")
    bpw = B // 32
    b0 = wid * bpw
    pltpu.sync_copy(uidx_hbm.at[pl.ds(b0, bpw)], idx_v)
    descs = [
        pltpu.async_copy(utab_hbm.at[idx_v.at[pl.ds(j * 128, 128)]],
                         out_v.at[pl.ds(j * 128, 128), :], sem)
        for j in range(bpw // 128)
    ]
    for c in descs:
        c.wait()
    pltpu.sync_copy(out_v, uemb_hbm.at[pl.ds(b0, bpw)])


def _user_gather(user_idx, user_table):
    bpw = B // 32
    mesh = plsc.VectorSubcoreMesh(core_axis_name="c", subcore_axis_name="s")
    f = pl.kernel(
        _user_body,
        out_type=jax.ShapeDtypeStruct((B, D), jnp.float32),
        mesh=mesh,
        scratch_types=[
            pltpu.VMEM((bpw, D), jnp.float32),
            pltpu.VMEM((bpw,), jnp.int32),
            pltpu.SemaphoreType.DMA,
        ],
        compiler_params=pltpu.CompilerParams(use_tc_tiling_on_sc=False),
    )
    return f(user_idx, user_table)


def _sc_gather(history, g, news_table, news_idx):
    bpw = B // 32
    mesh = plsc.VectorSubcoreMesh(core_axis_name="c", subcore_axis_name="s")
    f = pl.kernel(
        _sc_body,
        out_type=(
            jax.ShapeDtypeStruct((B, D), jnp.float32),
            jax.ShapeDtypeStruct((B, 16), jnp.float32),
            jax.ShapeDtypeStruct((B, D), jnp.float32),
        ),
        mesh=mesh,
        scratch_types=[
            pltpu.VMEM((16, L), jnp.int32),
            pltpu.VMEM((16, LP), jnp.float32),
            pltpu.VMEM((LP,), jnp.float32),
            pltpu.VMEM((2, LP, D), jnp.float32),
            pltpu.VMEM((bpw, D), jnp.float32),
            pltpu.VMEM((bpw, 16), jnp.float32),
            pltpu.VMEM((bpw,), jnp.int32),
            pltpu.SemaphoreType.DMA,
            pltpu.SemaphoreType.DMA,
            pltpu.SemaphoreType.DMA,
        ],
        compiler_params=pltpu.CompilerParams(use_tc_tiling_on_sc=False),
    )
    return f(history, g, news_table, news_idx)


_RB = 2048


def _final_body(ue_ref, hn_ref, den_ref, ie_ref, wut_ref, but_ref, wnt_ref,
                bnt_ref, out_ref):
    den = jnp.sum(den_ref[...], axis=1, keepdims=True)
    hr = hn_ref[...] * jnp.where(den > 0, 1.0 / den, 0.0)
    u = ue_ref[...] + hr
    ur = jax.nn.relu(
        lax.dot_general(u, wut_ref[...], (((1,), (1,)), ((), ())),
                        preferred_element_type=jnp.float32)
        + but_ref[...][None, :])
    nr = jax.nn.relu(
        lax.dot_general(ie_ref[...], wnt_ref[...], (((1,), (1,)), ((), ())),
                        preferred_element_type=jnp.float32)
        + bnt_ref[...][None, :])
    out_ref[...] = jax.nn.sigmoid(jnp.sum(ur * nr, axis=1))


def _final(user_emb, hist_num, den, id_emb, W_ut, b_ut, W_nt, b_nt):
    grid = B // _RB
    return pl.pallas_call(
        _final_body,
        grid=(grid,),
        in_specs=[
            pl.BlockSpec((_RB, D), lambda i: (i, 0)),
            pl.BlockSpec((_RB, D), lambda i: (i, 0)),
            pl.BlockSpec((_RB, 16), lambda i: (i, 0)),
            pl.BlockSpec((_RB, D), lambda i: (i, 0)),
            pl.BlockSpec((D, D), lambda i: (0, 0)),
            pl.BlockSpec((D,), lambda i: (0,)),
            pl.BlockSpec((D, D), lambda i: (0, 0)),
            pl.BlockSpec((D,), lambda i: (0,)),
        ],
        out_specs=pl.BlockSpec((_RB,), lambda i: (i,)),
        out_shape=jax.ShapeDtypeStruct((B,), jnp.float32),
    )(user_emb, hist_num, den, id_emb, W_ut, b_ut, W_nt, b_nt)


def kernel(user_idx, news_idx, history, user_table, news_table,
           W_ut, b_ut, W_nt, b_nt, W_a1, b_a1, W_a2, b_a2):
    news_t = news_table.T
    g = _score_table(news_t, W_a1, b_a1, W_a2, b_a2)
    hist_num, den, id_emb = _sc_gather(history, g, news_table, news_idx)
    user_emb = _user_gather(user_idx, user_table)
    return _final(user_emb, hist_num, den, id_emb, W_ut, b_ut, W_nt, b_nt)


# speedup vs baseline: 4.5475x; 1.0101x over previous
import jax
import jax.numpy as jnp
from jax import lax
from jax.experimental import pallas as pl
from jax.experimental.pallas import tpu as pltpu
from jax.experimental.pallas import tpu_sc as plsc

B = 16384
L = 200
D = 64
N = 1_000_000
LP = 208
NEG = -1e9

_BK = 8192


def _score_table_body(ntt_ref, w1_ref, b1_ref, w2_ref, b2_ref, out_ref):
    i = pl.program_id(0)
    x = ntt_ref[...]
    a = jnp.tanh(
        lax.dot_general(w1_ref[...], x, (((1,), (0,)), ((), ())),
                        preferred_element_type=jnp.float32)
        + b1_ref[...][:, None])
    g = lax.dot_general(w2_ref[...], a, (((1,), (0,)), ((), ())),
                        preferred_element_type=jnp.float32)
    g = (g + b2_ref[...][None, :]).reshape(_BK)
    nidx = lax.broadcasted_iota(jnp.int32, (_BK,), 0) + i * _BK
    out_ref[...] = jnp.where(nidx == 0, jnp.float32(NEG), g)


def _score_table(news_t, W_a1, b_a1, W_a2, b_a2):
    grid = pl.cdiv(N, _BK)
    return pl.pallas_call(
        _score_table_body,
        grid=(grid,),
        in_specs=[
            pl.BlockSpec((D, _BK), lambda i: (0, i)),
            pl.BlockSpec((D // 2, D), lambda i: (0, 0)),
            pl.BlockSpec((D // 2,), lambda i: (0,)),
            pl.BlockSpec((1, D // 2), lambda i: (0, 0)),
            pl.BlockSpec((1,), lambda i: (0,)),
        ],
        out_specs=pl.BlockSpec((_BK,), lambda i: (i,)),
        out_shape=jax.ShapeDtypeStruct((N,), jnp.float32),
    )(news_t, W_a1, b_a1, W_a2, b_a2)


def _sc_body(hist_hbm, g_hbm, news_hbm, nidx_hbm,
             hnum_hbm, den_hbm, iemb_hbm,
             hist_v, s2_v, w_v, rows_v, out_v, den_v, ie_v, idx_v,
             sem_s0, sem_s1, sem_r0, sem_r1, sem_h, sem_i):
    nc = 2
    wid = lax.axis_index("s") * nc + lax.axis_index("c")
    bpw = B // 32
    b0 = wid * bpw
    cc = 16
    nchunk = bpw // cc
    sem_s = (sem_s0, sem_s1)
    sem_r = (sem_r0, sem_r1)

    zf = jnp.zeros((16,), jnp.float32)
    for hh in range(2):
        for r in range(cc):
            s2_v[hh, r, pl.ds(192, 16)] = jnp.full((16,), NEG, jnp.float32)
    for nb in range(2):
        for r in range(L, LP):
            for q in range(D // 16):
                rows_v[nb, r, pl.ds(16 * q, 16)] = zf

    pltpu.sync_copy(nidx_hbm.at[pl.ds(b0, bpw)], idx_v)
    id_descs = [
        pltpu.async_copy(news_hbm.at[idx_v.at[pl.ds(j * 128, 128)]],
                         ie_v.at[pl.ds(j * 128, 128), :], sem_i)
        for j in range(bpw // 128)
    ]

    def _issue_scores(hh, r):
        pltpu.async_copy(g_hbm.at[hist_v.at[hh, r, pl.ds(0, 104)]],
                         s2_v.at[hh, r, pl.ds(0, 104)], sem_s[hh])
        pltpu.async_copy(g_hbm.at[hist_v.at[hh, r, pl.ds(104, 96)]],
                         s2_v.at[hh, r, pl.ds(104, 96)], sem_s[hh])

    def _drain_scores(hh, r):
        pltpu.make_async_copy(g_hbm.at[pl.ds(0, 104)],
                              s2_v.at[hh, r, pl.ds(0, 104)], sem_s[hh]).wait()
        pltpu.make_async_copy(g_hbm.at[pl.ds(0, 96)],
                              s2_v.at[hh, r, pl.ds(104, 96)], sem_s[hh]).wait()

    def _issue_rows(hh, r, nb):
        pltpu.async_copy(news_hbm.at[hist_v.at[hh, r, pl.ds(0, 104)]],
                         rows_v.at[nb, pl.ds(0, 104), :], sem_r[nb])
        pltpu.async_copy(news_hbm.at[hist_v.at[hh, r, pl.ds(104, 96)]],
                         rows_v.at[nb, pl.ds(104, 96), :], sem_r[nb])

    def _drain_rows(nb):
        pltpu.make_async_copy(news_hbm.at[pl.ds(0, 104), :],
                              rows_v.at[nb, pl.ds(0, 104), :],
                              sem_r[nb]).wait()
        pltpu.make_async_copy(news_hbm.at[pl.ds(0, 96), :],
                              rows_v.at[nb, pl.ds(104, 96), :],
                              sem_r[nb]).wait()

    pltpu.async_copy(hist_hbm.at[pl.ds(b0, cc), :], hist_v.at[0],
                     sem_h).wait()
    for r in range(cc):
        _issue_scores(0, r)

    @pl.loop(0, nchunk // 2)
    def _pair(i):
        for hh in range(2):
            ci = 2 * i + hh

            @pl.when(ci + 1 < nchunk)
            def _():
                pltpu.async_copy(
                    hist_hbm.at[pl.ds(b0 + (ci + 1) * cc, cc), :],
                    hist_v.at[1 - hh], sem_h)

            _issue_rows(hh, 0, 0)

            @pl.loop(0, cc // 2)
            def _bpair(j):
                for par in range(2):
                    r = 2 * j + par
                    bl = ci * cc + r

                    @pl.when(r < cc - 1)
                    def _():
                        _issue_rows(hh, r + 1, 1 - par)

                    _drain_scores(hh, r)
                    den = jnp.zeros((16,), jnp.float32)
                    for k in range(LP // 16):
                        e = jnp.exp(s2_v[hh, r, pl.ds(16 * k, 16)])
                        w_v[pl.ds(16 * k, 16)] = e
                        den = den + e
                    _drain_rows(par)

                    z = jnp.zeros((16,), jnp.float32)

                    @pl.loop(0, LP // 16, init_carry=(z, z, z, z))
                    def _acc(k, carry):
                        wv = w_v[pl.ds(16 * k, 16)]
                        for jj in range(16):
                            wl = wv[jj]
                            l = 16 * k + jj
                            carry = tuple(
                                c + rows_v[par, l, pl.ds(16 * q, 16)] * wl
                                for q, c in enumerate(carry))
                        return carry

                    a0, a1, a2, a3 = _acc
                    out_v[bl, pl.ds(0, 16)] = a0
                    out_v[bl, pl.ds(16, 16)] = a1
                    out_v[bl, pl.ds(32, 16)] = a2
                    out_v[bl, pl.ds(48, 16)] = a3
                    den_v[bl, :] = den

            @pl.when(ci + 1 < nchunk)
            def _():
                pltpu.make_async_copy(hist_hbm.at[pl.ds(0, cc), :],
                                      hist_v.at[1 - hh], sem_h).wait()
                for r in range(cc):
                    _issue_scores(1 - hh, r)

    pltpu.sync_copy(out_v, hnum_hbm.at[pl.ds(b0, bpw)])
    pltpu.sync_copy(den_v, den_hbm.at[pl.ds(b0, bpw)])
    for c in id_descs:
        c.wait()
    pltpu.sync_copy(ie_v, iemb_hbm.at[pl.ds(b0, bpw)])


def _user_body(uidx_hbm, utab_hbm, uemb_hbm, out_v, idx_v, sem):
    nc = 2
    wid = lax.axis_index("s") * nc + lax.axis_index("c")
    bpw = B // 32
    b0 = wid * bpw
    pltpu.sync_copy(uidx_hbm.at[pl.ds(b0, bpw)], idx_v)
    descs = [
        pltpu.async_copy(utab_hbm.at[idx_v.at[pl.ds(j * 128, 128)]],
                         out_v.at[pl.ds(j * 128, 128), :], sem)
        for j in range(bpw // 128)
    ]
    for c in descs:
        c.wait()
    pltpu.sync_copy(out_v, uemb_hbm.at[pl.ds(b0, bpw)])


def _user_gather(user_idx, user_table):
    bpw = B // 32
    mesh = plsc.VectorSubcoreMesh(core_axis_name="c", subcore_axis_name="s")
    f = pl.kernel(
        _user_body,
        out_type=jax.ShapeDtypeStruct((B, D), jnp.float32),
        mesh=mesh,
        scratch_types=[
            pltpu.VMEM((bpw, D), jnp.float32),
            pltpu.VMEM((bpw,), jnp.int32),
            pltpu.SemaphoreType.DMA,
        ],
        compiler_params=pltpu.CompilerParams(use_tc_tiling_on_sc=False),
    )
    return f(user_idx, user_table)


def _sc_gather(history, g, news_table, news_idx):
    bpw = B // 32
    mesh = plsc.VectorSubcoreMesh(core_axis_name="c", subcore_axis_name="s")
    f = pl.kernel(
        _sc_body,
        out_type=(
            jax.ShapeDtypeStruct((B, D), jnp.float32),
            jax.ShapeDtypeStruct((B, 16), jnp.float32),
            jax.ShapeDtypeStruct((B, D), jnp.float32),
        ),
        mesh=mesh,
        scratch_types=[
            pltpu.VMEM((2, 16, L), jnp.int32),
            pltpu.VMEM((2, 16, LP), jnp.float32),
            pltpu.VMEM((LP,), jnp.float32),
            pltpu.VMEM((2, LP, D), jnp.float32),
            pltpu.VMEM((bpw, D), jnp.float32),
            pltpu.VMEM((bpw, 16), jnp.float32),
            pltpu.VMEM((bpw, D), jnp.float32),
            pltpu.VMEM((bpw,), jnp.int32),
            pltpu.SemaphoreType.DMA,
            pltpu.SemaphoreType.DMA,
            pltpu.SemaphoreType.DMA,
            pltpu.SemaphoreType.DMA,
            pltpu.SemaphoreType.DMA,
            pltpu.SemaphoreType.DMA,
        ],
        compiler_params=pltpu.CompilerParams(use_tc_tiling_on_sc=False),
    )
    return f(history, g, news_table, news_idx)


_RB = 2048


def _final_body(ue_ref, hn_ref, den_ref, ie_ref, wut_ref, but_ref, wnt_ref,
                bnt_ref, out_ref):
    den = jnp.sum(den_ref[...], axis=1, keepdims=True)
    hr = hn_ref[...] * jnp.where(den > 0, 1.0 / den, 0.0)
    u = ue_ref[...] + hr
    ur = jax.nn.relu(
        lax.dot_general(u, wut_ref[...], (((1,), (1,)), ((), ())),
                        preferred_element_type=jnp.float32)
        + but_ref[...][None, :])
    nr = jax.nn.relu(
        lax.dot_general(ie_ref[...], wnt_ref[...], (((1,), (1,)), ((), ())),
                        preferred_element_type=jnp.float32)
        + bnt_ref[...][None, :])
    out_ref[...] = jax.nn.sigmoid(jnp.sum(ur * nr, axis=1))


def _final(user_emb, hist_num, den, id_emb, W_ut, b_ut, W_nt, b_nt):
    grid = B // _RB
    return pl.pallas_call(
        _final_body,
        grid=(grid,),
        in_specs=[
            pl.BlockSpec((_RB, D), lambda i: (i, 0)),
            pl.BlockSpec((_RB, D), lambda i: (i, 0)),
            pl.BlockSpec((_RB, 16), lambda i: (i, 0)),
            pl.BlockSpec((_RB, D), lambda i: (i, 0)),
            pl.BlockSpec((D, D), lambda i: (0, 0)),
            pl.BlockSpec((D,), lambda i: (0,)),
            pl.BlockSpec((D, D), lambda i: (0, 0)),
            pl.BlockSpec((D,), lambda i: (0,)),
        ],
        out_specs=pl.BlockSpec((_RB,), lambda i: (i,)),
        out_shape=jax.ShapeDtypeStruct((B,), jnp.float32),
    )(user_emb, hist_num, den, id_emb, W_ut, b_ut, W_nt, b_nt)


def kernel(user_idx, news_idx, history, user_table, news_table,
           W_ut, b_ut, W_nt, b_nt, W_a1, b_a1, W_a2, b_a2):
    news_t = news_table.T
    g = _score_table(news_t, W_a1, b_a1, W_a2, b_a2)
    hist_num, den, id_emb = _sc_gather(history, g, news_table, news_idx)
    user_emb = _user_gather(user_idx, user_table)
    return _final(user_emb, hist_num, den, id_emb, W_ut, b_ut, W_nt, b_nt)
